# baseline (speedup 1.0000x reference)
import functools

import jax
import jax.numpy as jnp
from jax.experimental import pallas as pl
from jax.experimental.pallas import tpu as pltpu


BLK = 2048
PAD_N = 640


def _body(x_ref, raw_ref, wf_ref, b_ref, out_ref, idx_ref, wfs_ref, *,
          n_tiles, n_classes):
    d_model = x_ref.shape[1]
    n_flat = n_tiles * n_classes

    @pl.when(pl.program_id(0) == 0)
    def _init():
        rawv = raw_ref[:, :]
        sigs = jnp.where(rawv > 0.3, 1.0, jnp.where(rawv < -0.3, -1.0, 0.0))
        wfs_ref[0:n_flat, :] = wf_ref[:, :]
        wfs_ref[n_flat:n_flat + n_tiles, :] = sigs
        wfs_ref[n_flat + n_tiles:, :] = jnp.zeros(
            (PAD_N - n_flat - n_tiles, d_model), jnp.float32)

    xb = x_ref[:, :]
    alls = jax.lax.dot_general(
        xb, wfs_ref[:, :], (((1,), (1,)), ((), ())),
        preferred_element_type=jnp.float32)

    scores_t = alls[:, n_flat:n_flat + n_tiles].T
    iota_st = jax.lax.broadcasted_iota(jnp.int32, scores_t.shape, 0)
    m_t = jnp.max(scores_t, axis=0, keepdims=True)
    idx = jnp.min(jnp.where(scores_t == m_t, iota_st, n_tiles), axis=0)
    onehot_t = (iota_st == idx[None, :]).astype(jnp.float32)

    gri = jax.lax.broadcasted_iota(jnp.int32, (n_tiles, n_flat), 0)
    gli = jax.lax.broadcasted_iota(jnp.int32, (n_tiles, n_flat), 1)
    grp = (gli // n_classes == gri).astype(jnp.float32)
    mask = jax.lax.dot_general(
        onehot_t, grp, (((0,), (0,)), ((), ())),
        preferred_element_type=jnp.float32)
    masked = alls[:, :n_flat] * mask
    rowi = jax.lax.broadcasted_iota(jnp.int32, (n_flat, n_classes), 0)
    coli = jax.lax.broadcasted_iota(jnp.int32, (n_flat, n_classes), 1)
    fold = (rowi % n_classes == coli).astype(jnp.float32)
    logits = jax.lax.dot_general(
        masked, fold, (((1,), (0,)), ((), ())),
        preferred_element_type=jnp.float32)

    bsel = jax.lax.dot_general(
        onehot_t, b_ref[:, :], (((0,), (0,)), ((), ())),
        preferred_element_type=jnp.float32)

    out_ref[:, :] = logits + bsel
    idx_ref[0, 0, :] = idx


@jax.jit
def kernel(x, raw, W, b):
    n_tok, d_model = x.shape
    n_tiles, n_classes, _ = W.shape
    wf = W.reshape(n_tiles * n_classes, d_model)
    grid = n_tok // BLK

    logits, idx3 = pl.pallas_call(
        functools.partial(_body, n_tiles=n_tiles, n_classes=n_classes),
        grid=(grid,),
        in_specs=[
            pl.BlockSpec((BLK, d_model), lambda i: (i, 0)),
            pl.BlockSpec((n_tiles, d_model), lambda i: (0, 0)),
            pl.BlockSpec((n_tiles * n_classes, d_model), lambda i: (0, 0)),
            pl.BlockSpec((n_tiles, n_classes), lambda i: (0, 0)),
        ],
        out_specs=[
            pl.BlockSpec((BLK, n_classes), lambda i: (i, 0)),
            pl.BlockSpec((1, 1, BLK), lambda i: (i, 0, 0)),
        ],
        out_shape=[
            jax.ShapeDtypeStruct((n_tok, n_classes), jnp.float32),
            jax.ShapeDtypeStruct((grid, 1, BLK), jnp.int32),
        ],
        scratch_shapes=[pltpu.VMEM((PAD_N, d_model), jnp.float32)],
    )(x, raw, wf, b)

    return logits, idx3.reshape(n_tok)


# speedup vs baseline: 1.2495x; 1.2495x over previous
import functools

import jax
import jax.numpy as jnp
from jax.experimental import pallas as pl
from jax.experimental.pallas import tpu as pltpu


BLK = 2048


def _body(x_ref, raw_ref, wf_ref, b_ref, out_ref, idx_ref, *, n_tiles, n_classes):
    xb = x_ref[:, :]
    rawv = raw_ref[:, :]
    sigs = jnp.where(rawv > 0.3, 1.0, jnp.where(rawv < -0.3, -1.0, 0.0))

    scores = jax.lax.dot_general(
        xb, sigs, (((1,), (1,)), ((), ())),
        preferred_element_type=jnp.float32)
    scores_t = scores.T
    iota_st = jax.lax.broadcasted_iota(jnp.int32, scores_t.shape, 0)
    m_t = jnp.max(scores_t, axis=0, keepdims=True)
    idx = jnp.min(jnp.where(scores_t == m_t, iota_st, n_tiles), axis=0)
    onehot_t = (iota_st == idx[None, :]).astype(jnp.float32)

    alll = jax.lax.dot_general(
        xb, wf_ref[:, :], (((1,), (1,)), ((), ())),
        preferred_element_type=jnp.float32)
    gri = jax.lax.broadcasted_iota(jnp.int32, (n_tiles, n_tiles * n_classes), 0)
    gli = jax.lax.broadcasted_iota(jnp.int32, (n_tiles, n_tiles * n_classes), 1)
    grp = (gli // n_classes == gri).astype(jnp.float32)
    mask = jax.lax.dot_general(
        onehot_t, grp, (((0,), (0,)), ((), ())),
        preferred_element_type=jnp.float32)
    masked = alll * mask
    rowi = jax.lax.broadcasted_iota(jnp.int32, (n_tiles * n_classes, n_classes), 0)
    coli = jax.lax.broadcasted_iota(jnp.int32, (n_tiles * n_classes, n_classes), 1)
    fold = (rowi % n_classes == coli).astype(jnp.float32)
    logits = jax.lax.dot_general(
        masked, fold, (((1,), (0,)), ((), ())),
        preferred_element_type=jnp.float32)

    bsel = jax.lax.dot_general(
        onehot_t, b_ref[:, :], (((0,), (0,)), ((), ())),
        preferred_element_type=jnp.float32)

    out_ref[:, :] = logits + bsel
    idx_ref[0, 0, :] = idx


@jax.jit
def kernel(x, raw, W, b):
    n_tok, d_model = x.shape
    n_tiles, n_classes, _ = W.shape
    wf = W.reshape(n_tiles * n_classes, d_model)
    grid = n_tok // BLK

    logits, idx3 = pl.pallas_call(
        functools.partial(_body, n_tiles=n_tiles, n_classes=n_classes),
        grid=(grid,),
        in_specs=[
            pl.BlockSpec((BLK, d_model), lambda i: (i, 0)),
            pl.BlockSpec((n_tiles, d_model), lambda i: (0, 0)),
            pl.BlockSpec((n_tiles * n_classes, d_model), lambda i: (0, 0)),
            pl.BlockSpec((n_tiles, n_classes), lambda i: (0, 0)),
        ],
        out_specs=[
            pl.BlockSpec((BLK, n_classes), lambda i: (i, 0)),
            pl.BlockSpec((1, 1, BLK), lambda i: (i, 0, 0)),
        ],
        out_shape=[
            jax.ShapeDtypeStruct((n_tok, n_classes), jnp.float32),
            jax.ShapeDtypeStruct((grid, 1, BLK), jnp.int32),
        ],
        compiler_params=pltpu.CompilerParams(
            dimension_semantics=("parallel",)),
    )(x, raw, wf, b)

    return logits, idx3.reshape(n_tok)


# speedup vs baseline: 1.2523x; 1.0022x over previous
import functools

import jax
import jax.numpy as jnp
from jax.experimental import pallas as pl
from jax.experimental.pallas import tpu as pltpu


BLK = 2048


def _body(x_ref, raw_ref, wf_ref, b_ref, out_ref, idx_ref, *, n_tiles, n_classes):
    xb = x_ref[:, :]
    rawv = raw_ref[:, :]
    sigs = jnp.where(rawv > 0.3, 1.0, jnp.where(rawv < -0.3, -1.0, 0.0))

    scores = jax.lax.dot_general(
        xb, sigs, (((1,), (1,)), ((), ())),
        preferred_element_type=jnp.float32)
    scores_t = scores.T
    iota_st = jax.lax.broadcasted_iota(jnp.int32, scores_t.shape, 0)
    m_t = jnp.max(scores_t, axis=0, keepdims=True)
    idx = jnp.min(jnp.where(scores_t == m_t, iota_st, n_tiles), axis=0)
    onehot_t = (iota_st == idx[None, :]).astype(jnp.float32)

    alll = jax.lax.dot_general(
        xb, wf_ref[:, :], (((1,), (1,)), ((), ())),
        preferred_element_type=jnp.float32)
    gri = jax.lax.broadcasted_iota(jnp.int32, (n_tiles, n_tiles * n_classes), 0)
    gli = jax.lax.broadcasted_iota(jnp.int32, (n_tiles, n_tiles * n_classes), 1)
    grp = (gli // n_classes == gri).astype(jnp.float32)
    mask = jax.lax.dot_general(
        onehot_t, grp, (((0,), (0,)), ((), ())),
        preferred_element_type=jnp.float32)
    masked = alll * mask
    rowi = jax.lax.broadcasted_iota(jnp.int32, (n_tiles * n_classes, n_classes), 0)
    coli = jax.lax.broadcasted_iota(jnp.int32, (n_tiles * n_classes, n_classes), 1)
    fold = (rowi % n_classes == coli).astype(jnp.float32)
    logits = jax.lax.dot_general(
        masked, fold, (((1,), (0,)), ((), ())),
        preferred_element_type=jnp.float32)

    bsel = jax.lax.dot_general(
        onehot_t, b_ref[:, :], (((0,), (0,)), ((), ())),
        preferred_element_type=jnp.float32)

    out_ref[:, :] = logits + bsel
    idx_ref[:] = idx


@jax.jit
def kernel(x, raw, W, b):
    n_tok, d_model = x.shape
    n_tiles, n_classes, _ = W.shape
    wf = W.reshape(n_tiles * n_classes, d_model)
    grid = n_tok // BLK

    logits, idx3 = pl.pallas_call(
        functools.partial(_body, n_tiles=n_tiles, n_classes=n_classes),
        grid=(grid,),
        in_specs=[
            pl.BlockSpec((BLK, d_model), lambda i: (i, 0)),
            pl.BlockSpec((n_tiles, d_model), lambda i: (0, 0)),
            pl.BlockSpec((n_tiles * n_classes, d_model), lambda i: (0, 0)),
            pl.BlockSpec((n_tiles, n_classes), lambda i: (0, 0)),
        ],
        out_specs=[
            pl.BlockSpec((BLK, n_classes), lambda i: (i, 0)),
            pl.BlockSpec((BLK,), lambda i: (i,)),
        ],
        out_shape=[
            jax.ShapeDtypeStruct((n_tok, n_classes), jnp.float32),
            jax.ShapeDtypeStruct((n_tok,), jnp.int32),
        ],
        compiler_params=pltpu.CompilerParams(
            dimension_semantics=("parallel",)),
    )(x, raw, wf, b)

    return logits, idx3


# speedup vs baseline: 1.2561x; 1.0031x over previous
import functools

import jax
import jax.numpy as jnp
from jax.experimental import pallas as pl
from jax.experimental.pallas import tpu as pltpu


BLK = 2048


def _body(x_ref, raw_ref, wf_ref, b_ref, out_ref, idx_ref, *, n_tiles, n_classes):
    xb = x_ref[:, :]
    rawv = raw_ref[:, :]
    sigs = jnp.where(rawv > 0.3, 1.0, jnp.where(rawv < -0.3, -1.0, 0.0))

    scores = jax.lax.dot_general(
        xb, sigs, (((1,), (1,)), ((), ())),
        preferred_element_type=jnp.float32)
    scores_t = scores.T
    iota_st = jax.lax.broadcasted_iota(jnp.int32, scores_t.shape, 0)
    m_t = jnp.max(scores_t, axis=0, keepdims=True)
    idx = jnp.min(jnp.where(scores_t == m_t, iota_st, n_tiles), axis=0)
    onehot_t = (iota_st == idx[None, :]).astype(jnp.float32)

    alll = jax.lax.dot_general(
        xb, wf_ref[:, :], (((1,), (1,)), ((), ())),
        preferred_element_type=jnp.float32)

    tvec = jax.lax.broadcasted_iota(
        jnp.int32, (n_tiles, 8), 0).astype(jnp.float32)
    idxf = jax.lax.dot_general(
        onehot_t, tvec, (((0,), (0,)), ((), ())),
        preferred_element_type=jnp.float32)[:, :1]

    s0 = alll[:, 0 * 128:1 * 128]
    s1 = alll[:, 1 * 128:2 * 128]
    s2 = alll[:, 2 * 128:3 * 128]
    s3 = alll[:, 3 * 128:4 * 128]
    m01 = jnp.where(idxf < 2.0, s0, s1)
    m23 = jnp.where(idxf < 6.0, s2, s3)
    u = jnp.where(idxf < 4.0, m01, m23)
    odd = idxf - 2.0 * jnp.floor(idxf * 0.5)
    logits = jnp.where(odd < 0.5, u[:, :n_classes], u[:, n_classes:])

    bsel = jax.lax.dot_general(
        onehot_t, b_ref[:, :], (((0,), (0,)), ((), ())),
        preferred_element_type=jnp.float32)

    out_ref[:, :] = logits + bsel
    idx_ref[:] = idx


@jax.jit
def kernel(x, raw, W, b):
    n_tok, d_model = x.shape
    n_tiles, n_classes, _ = W.shape
    wf = W.reshape(n_tiles * n_classes, d_model)
    grid = n_tok // BLK

    logits, idx3 = pl.pallas_call(
        functools.partial(_body, n_tiles=n_tiles, n_classes=n_classes),
        grid=(grid,),
        in_specs=[
            pl.BlockSpec((BLK, d_model), lambda i: (i, 0)),
            pl.BlockSpec((n_tiles, d_model), lambda i: (0, 0)),
            pl.BlockSpec((n_tiles * n_classes, d_model), lambda i: (0, 0)),
            pl.BlockSpec((n_tiles, n_classes), lambda i: (0, 0)),
        ],
        out_specs=[
            pl.BlockSpec((BLK, n_classes), lambda i: (i, 0)),
            pl.BlockSpec((BLK,), lambda i: (i,)),
        ],
        out_shape=[
            jax.ShapeDtypeStruct((n_tok, n_classes), jnp.float32),
            jax.ShapeDtypeStruct((n_tok,), jnp.int32),
        ],
        compiler_params=pltpu.CompilerParams(
            dimension_semantics=("parallel",)),
    )(x, raw, wf, b)

    return logits, idx3
